# baseline (speedup 1.0000x reference)
import functools

import jax
import jax.numpy as jnp
from jax import lax
from jax.experimental import pallas as pl
from jax.experimental.pallas import tpu as pltpu
from jax.experimental.pallas import tpu_sc as plsc

B = 4096
T = 50
V = 100000
D = 128
E = 16
K = 2
F_OUT = 128
PAD = 0

SLICES = (2560, 1536)
NW = 32
CHUNK = 400


def _gather_tokens(idx, table, bs):
    n_idx = bs * T
    b_per_w = n_idx // NW
    n_chunks = b_per_w // CHUNK

    def _sc_gather(idx_hbm, table_hbm, out_hbm, idx_v, rows0, rows1,
                   gs0, gs1, ws0, ws1):
        wid = lax.axis_index("s") * 2 + lax.axis_index("c")
        base = wid * b_per_w
        pltpu.sync_copy(idx_hbm.at[pl.ds(base, b_per_w)], idx_v)

        bufs, gsems, wsems = (rows0, rows1), (gs0, gs1), (ws0, ws1)
        gh = [None, None]
        wh = [None, None]

        def start_gather(c):
            b = c & 1
            gh[b] = pltpu.async_copy(
                table_hbm.at[idx_v.at[pl.ds(c * CHUNK, CHUNK)]], bufs[b],
                gsems[b])

        start_gather(0)
        for c in range(n_chunks):
            b = c & 1
            gh[b].wait()
            if c + 1 < n_chunks:
                if c >= 1:
                    wh[1 - b].wait()
                start_gather(c + 1)
            wh[b] = pltpu.async_copy(
                bufs[b], out_hbm.at[pl.ds(base + c * CHUNK, CHUNK)],
                wsems[b])
        wh[(n_chunks - 1) & 1].wait()

    mesh = plsc.VectorSubcoreMesh(core_axis_name="c", subcore_axis_name="s")
    k = functools.partial(
        pl.kernel,
        mesh=mesh,
        out_type=jax.ShapeDtypeStruct((n_idx, D), jnp.float32),
        scratch_types=[
            pltpu.VMEM((b_per_w,), jnp.int32),
            pltpu.VMEM((CHUNK, D), jnp.float32),
            pltpu.VMEM((CHUNK, D), jnp.float32),
            pltpu.SemaphoreType.DMA,
            pltpu.SemaphoreType.DMA,
            pltpu.SemaphoreType.DMA,
            pltpu.SemaphoreType.DMA,
        ],
    )(_sc_gather)
    return k(idx, table)


R = 512


def _tc_moe(samT_ref, tok_ref, wk_ref, q_ref, wg_ref, bg_ref, wt_ref,
            bt_ref, u_ref, out_ref, es_scr):
    proj = jnp.concatenate(
        [jax.lax.dot_general(q_ref[pl.ds(e, 1), :], wk_ref[e],
                             (((1,), (0,)), ((), ())),
                             preferred_element_type=jnp.float32)
         for e in range(E)], axis=0)
    cat = jnp.concatenate([proj, wg_ref[...]], axis=0)

    samT = samT_ref[...]
    zero_er = jnp.zeros((E, R), jnp.float32)
    sumT = zero_er
    g_acc = zero_er
    for t in range(T):
        s_t = jax.lax.dot_general(cat, tok_ref[t], (((1,), (1,)), ((), ())),
                                  preferred_element_type=jnp.float32)
        mask_t = samT[t:t + 1, :] == PAD
        es_t = jnp.where(mask_t, 0.0, jnp.exp(s_t[:E]))
        sumT = sumT + es_t
        g_acc = g_acc + s_t[E:]
        es_scr[t] = es_t

    bg_col = bg_ref[...]
    g = g_acc * (1.0 / T) + bg_col

    eids = lax.broadcasted_iota(jnp.int32, (E, R), 0)
    m1 = jnp.max(g, axis=0, keepdims=True)
    i1 = jnp.min(jnp.where(g == m1, eids, E), axis=0, keepdims=True)
    g2 = jnp.where(eids == i1, -jnp.inf, g)
    m2 = jnp.max(g2, axis=0, keepdims=True)
    i2 = jnp.min(jnp.where(g2 == m2, eids, E), axis=0, keepdims=True)
    e2 = jnp.exp(m2 - m1)
    w1 = 1.0 / (1.0 + e2)
    w2 = e2 * w1
    full = jnp.where(eids == i1, w1, 0.0) + jnp.where(eids == i2, w2, 0.0)
    fulln = full / jnp.maximum(sumT, 1e-30)

    wrows = [jnp.sum(es_scr[t] * fulln, axis=0, keepdims=True)
             for t in range(T)]
    wts = jnp.transpose(jnp.concatenate(wrows, axis=0))

    pooled = jnp.zeros((R, D), jnp.float32)
    for t in range(T):
        pooled = pooled + wts[:, t:t + 1] * tok_ref[t]

    z = jax.lax.dot_general(pooled, wt_ref[...], (((1,), (1,)), ((), ())),
                            preferred_element_type=jnp.float32)
    z = z + bt_ref[...]
    out_ref[...] = z * jax.nn.sigmoid(z) * u_ref[...]


def _moe_dense(samT, tokT, Wk, q, Wg, bg, Wt, bt, u):
    bs = tokT.shape[1]
    grid = (bs // R,)
    return pl.pallas_call(
        _tc_moe,
        grid=grid,
        in_specs=[
            pl.BlockSpec((T, R), lambda i: (0, i)),
            pl.BlockSpec((T, R, D), lambda i: (0, i, 0)),
            pl.BlockSpec((E, D, D), lambda i: (0, 0, 0)),
            pl.BlockSpec((E, D), lambda i: (0, 0)),
            pl.BlockSpec((E, D), lambda i: (0, 0)),
            pl.BlockSpec((E, 1), lambda i: (0, 0)),
            pl.BlockSpec((F_OUT, D), lambda i: (0, 0)),
            pl.BlockSpec((1, F_OUT), lambda i: (0, 0)),
            pl.BlockSpec((1, 1), lambda i: (0, 0)),
        ],
        out_specs=pl.BlockSpec((R, F_OUT), lambda i: (i, 0)),
        out_shape=jax.ShapeDtypeStruct((bs, F_OUT), jnp.float32),
        scratch_shapes=[pltpu.VMEM((T, E, R), jnp.float32)],
    )(samT, tokT, Wk, q, Wg, bg, Wt, bt, u)


def kernel(sample, table, Wk, q, Wg, bg, Wt, bt, use_text_moe):
    samT = jnp.transpose(sample.astype(jnp.int32))
    bg_c = bg.reshape(E, 1)
    bt_r = bt.reshape(1, F_OUT)
    u = jnp.asarray(use_text_moe, jnp.float32).reshape(1, 1)
    outs = []
    off = 0
    for bs in SLICES:
        samT_s = samT[:, off:off + bs]
        tok_s = _gather_tokens(samT_s.reshape(bs * T), table, bs)
        outs.append(_moe_dense(samT_s, tok_s.reshape(T, bs, D),
                               Wk, q, Wg, bg_c, Wt, bt_r, u))
        off += bs
    return jnp.concatenate(outs, axis=0)


# speedup vs baseline: 1.0360x; 1.0360x over previous
import functools

import jax
import jax.numpy as jnp
from jax import lax
from jax.experimental import pallas as pl
from jax.experimental.pallas import tpu as pltpu
from jax.experimental.pallas import tpu_sc as plsc

B = 4096
T = 50
V = 100000
D = 128
E = 16
K = 2
F_OUT = 128
PAD = 0

SLICES = (2048, 2048)
NW = 32
CHUNK = 400


def _gather_tokens(idx, table, bs):
    n_idx = bs * T
    b_per_w = n_idx // NW
    n_chunks = b_per_w // CHUNK

    def _sc_gather(idx_hbm, table_hbm, out_hbm, idx_v, rows0, rows1,
                   gs0, gs1, ws0, ws1):
        wid = lax.axis_index("s") * 2 + lax.axis_index("c")
        base = wid * b_per_w
        pltpu.sync_copy(idx_hbm.at[pl.ds(base, b_per_w)], idx_v)

        bufs, gsems, wsems = (rows0, rows1), (gs0, gs1), (ws0, ws1)
        gh = [None, None]
        wh = [None, None]

        def start_gather(c):
            b = c & 1
            gh[b] = pltpu.async_copy(
                table_hbm.at[idx_v.at[pl.ds(c * CHUNK, CHUNK)]], bufs[b],
                gsems[b])

        start_gather(0)
        for c in range(n_chunks):
            b = c & 1
            gh[b].wait()
            if c + 1 < n_chunks:
                if c >= 1:
                    wh[1 - b].wait()
                start_gather(c + 1)
            wh[b] = pltpu.async_copy(
                bufs[b], out_hbm.at[pl.ds(base + c * CHUNK, CHUNK)],
                wsems[b])
        wh[(n_chunks - 1) & 1].wait()

    mesh = plsc.VectorSubcoreMesh(core_axis_name="c", subcore_axis_name="s")
    k = functools.partial(
        pl.kernel,
        mesh=mesh,
        out_type=jax.ShapeDtypeStruct((n_idx, D), jnp.float32),
        scratch_types=[
            pltpu.VMEM((b_per_w,), jnp.int32),
            pltpu.VMEM((CHUNK, D), jnp.float32),
            pltpu.VMEM((CHUNK, D), jnp.float32),
            pltpu.SemaphoreType.DMA,
            pltpu.SemaphoreType.DMA,
            pltpu.SemaphoreType.DMA,
            pltpu.SemaphoreType.DMA,
        ],
    )(_sc_gather)
    return k(idx, table)


R = 512


def _tc_moe(samT_ref, tok_ref, wk_ref, q_ref, wg_ref, bg_ref, wt_ref,
            bt_ref, u_ref, out_ref, es_scr):
    proj = jnp.concatenate(
        [jax.lax.dot_general(q_ref[pl.ds(e, 1), :], wk_ref[e],
                             (((1,), (0,)), ((), ())),
                             preferred_element_type=jnp.float32)
         for e in range(E)], axis=0)
    cat = jnp.concatenate([proj, wg_ref[...]], axis=0)

    samT = samT_ref[...]
    zero_er = jnp.zeros((E, R), jnp.float32)
    sumT = zero_er
    g_acc = zero_er
    for t in range(T):
        s_t = jax.lax.dot_general(cat, tok_ref[t], (((1,), (1,)), ((), ())),
                                  preferred_element_type=jnp.float32)
        mask_t = samT[t:t + 1, :] == PAD
        es_t = jnp.where(mask_t, 0.0, jnp.exp(s_t[:E]))
        sumT = sumT + es_t
        g_acc = g_acc + s_t[E:]
        es_scr[t] = es_t

    bg_col = bg_ref[...]
    g = g_acc * (1.0 / T) + bg_col

    eids = lax.broadcasted_iota(jnp.int32, (E, R), 0)
    m1 = jnp.max(g, axis=0, keepdims=True)
    i1 = jnp.min(jnp.where(g == m1, eids, E), axis=0, keepdims=True)
    g2 = jnp.where(eids == i1, -jnp.inf, g)
    m2 = jnp.max(g2, axis=0, keepdims=True)
    i2 = jnp.min(jnp.where(g2 == m2, eids, E), axis=0, keepdims=True)
    e2 = jnp.exp(m2 - m1)
    w1 = 1.0 / (1.0 + e2)
    w2 = e2 * w1
    full = jnp.where(eids == i1, w1, 0.0) + jnp.where(eids == i2, w2, 0.0)
    fulln = full / jnp.maximum(sumT, 1e-30)

    wrows = [jnp.sum(es_scr[t] * fulln, axis=0, keepdims=True)
             for t in range(T)]
    wts = jnp.transpose(jnp.concatenate(wrows, axis=0))

    pooled = jnp.zeros((R, D), jnp.float32)
    for t in range(T):
        pooled = pooled + wts[:, t:t + 1] * tok_ref[t]

    z = jax.lax.dot_general(pooled, wt_ref[...], (((1,), (1,)), ((), ())),
                            preferred_element_type=jnp.float32)
    z = z + bt_ref[...]
    out_ref[...] = z * jax.nn.sigmoid(z) * u_ref[...]


def _moe_dense(samT, tokT, Wk, q, Wg, bg, Wt, bt, u):
    bs = tokT.shape[1]
    grid = (bs // R,)
    return pl.pallas_call(
        _tc_moe,
        grid=grid,
        in_specs=[
            pl.BlockSpec((T, R), lambda i: (0, i)),
            pl.BlockSpec((T, R, D), lambda i: (0, i, 0)),
            pl.BlockSpec((E, D, D), lambda i: (0, 0, 0)),
            pl.BlockSpec((E, D), lambda i: (0, 0)),
            pl.BlockSpec((E, D), lambda i: (0, 0)),
            pl.BlockSpec((E, 1), lambda i: (0, 0)),
            pl.BlockSpec((F_OUT, D), lambda i: (0, 0)),
            pl.BlockSpec((1, F_OUT), lambda i: (0, 0)),
            pl.BlockSpec((1, 1), lambda i: (0, 0)),
        ],
        out_specs=pl.BlockSpec((R, F_OUT), lambda i: (i, 0)),
        out_shape=jax.ShapeDtypeStruct((bs, F_OUT), jnp.float32),
        scratch_shapes=[pltpu.VMEM((T, E, R), jnp.float32)],
    )(samT, tokT, Wk, q, Wg, bg, Wt, bt, u)


def kernel(sample, table, Wk, q, Wg, bg, Wt, bt, use_text_moe):
    samT = jnp.transpose(sample.astype(jnp.int32))
    bg_c = bg.reshape(E, 1)
    bt_r = bt.reshape(1, F_OUT)
    u = jnp.asarray(use_text_moe, jnp.float32).reshape(1, 1)
    outs = []
    off = 0
    for bs in SLICES:
        samT_s = samT[:, off:off + bs]
        tok_s = _gather_tokens(samT_s.reshape(bs * T), table, bs)
        outs.append(_moe_dense(samT_s, tok_s.reshape(T, bs, D),
                               Wk, q, Wg, bg_c, Wt, bt_r, u))
        off += bs
    return jnp.concatenate(outs, axis=0)
